# Initial kernel scaffold; baseline (speedup 1.0000x reference)
import jax
import jax.numpy as jnp
from jax.experimental import pallas as pl


def kernel(x, edge_index, batch, W1, b1, W2, b2, W3, b3, L1w, L1b, L2w, L2b, L3w, L3b):
    raise NotImplementedError("write your pallas kernel here")


# speedup vs baseline: 2.9237x; 2.9237x over previous
import functools
import jax
import jax.numpy as jnp
from jax import lax
from jax.experimental import pallas as pl
from jax.experimental.pallas import tpu as pltpu
from jax.experimental.pallas import tpu_sc as plsc

NN = 100000
EE = 1600000
GG = 64
RB = 1024
NPAD = 98 * RB
NP16 = NN + 96
EP = 1638400
EROWS = EP // 128
TPR = NP16 // 16

_mesh = plsc.VectorSubcoreMesh(core_axis_name="c", subcore_axis_name="s")


_sc_params = pltpu.CompilerParams(use_tc_tiling_on_sc=False)


@functools.partial(
    pl.kernel, mesh=_mesh, compiler_params=_sc_params,
    out_type=jax.ShapeDtypeStruct((2, NP16, 16), jnp.float32),
    scratch_types=[
        pltpu.VMEM_SHARED((NP16, 16), jnp.float32),
        pltpu.VMEM((16, 128), jnp.int32),
        pltpu.VMEM((128, 16), jnp.float32),
    ],
)
def _deg_call(dst2d, ones_hbm, zeros_hbm, out, acc, dstv, onesv):
    cid = lax.axis_index("c")
    sid = lax.axis_index("s")
    pltpu.sync_copy(ones_hbm, onesv)
    pltpu.sync_copy(zeros_hbm, acc.at[pl.ds(sid * TPR, TPR)])
    plsc.subcore_barrier()
    wid = cid * 16 + sid
    base = wid * (EROWS // 32)

    def w(i, _):
        pltpu.sync_copy(dst2d.at[pl.ds(base + i * 16, 16)], dstv)
        for j in range(16):
            pltpu.sync_copy(onesv, acc.at[dstv.at[j]], add=True)
        return 0

    lax.fori_loop(0, 25, w, 0)
    plsc.subcore_barrier()
    pltpu.sync_copy(acc.at[pl.ds(sid * TPR, TPR)],
                    out.at[cid].at[pl.ds(sid * TPR, TPR)])


def _make_agg(C):
    @functools.partial(
        pl.kernel, mesh=_mesh, compiler_params=_sc_params,
        out_type=jax.ShapeDtypeStruct((C, NP16, 16), jnp.float32),
        scratch_types=[
            pltpu.VMEM_SHARED((NP16, 16), jnp.float32),
            pltpu.VMEM((8, 128), jnp.int32),
            pltpu.VMEM((8, 128), jnp.int32),
            pltpu.VMEM((1024, 16), jnp.float32),
            pltpu.SemaphoreType.DMA,
        ],
    )
    def _agg(ycm, src2d, dst2d, zeros_hbm, out, acc, srcv, dstv, rows, sem):
        cid = lax.axis_index("c")
        sid = lax.axis_index("s")
        nsl = pl.ds(sid * TPR, TPR)

        def chunk(k, _):
            @pl.when((k % 2) == cid)
            def _():
                pltpu.sync_copy(zeros_hbm, acc.at[nsl])
                plsc.subcore_barrier()
                base = sid * (EROWS // 16)

                def w(i, _):
                    off = base + i * 8
                    pltpu.sync_copy(src2d.at[pl.ds(off, 8)], srcv)
                    pltpu.sync_copy(dst2d.at[pl.ds(off, 8)], dstv)
                    hs = [pltpu.async_copy(ycm.at[k].at[srcv.at[j]],
                                           rows.at[pl.ds(j * 128, 128)], sem)
                          for j in range(8)]
                    for h in hs:
                        h.wait()
                    for j in range(8):
                        pltpu.sync_copy(rows.at[pl.ds(j * 128, 128)],
                                        acc.at[dstv.at[j]], add=True)
                    return 0

                lax.fori_loop(0, 100, w, 0)
                plsc.subcore_barrier()
                pltpu.sync_copy(acc.at[nsl], out.at[k].at[nsl])
            return 0

        lax.fori_loop(0, C, chunk, 0)

    return _agg


def _prep_body(d0, d1, xp, dinv_ref, xs_ref):
    deg = d0[...] + d1[...] + 1.0
    dv = lax.rsqrt(deg)
    dinv_ref[...] = dv
    xs_ref[...] = xp[...] * dv


def _l1_body(a1, xs, dinv, w1, b1, w2, y2_ref):
    pre = dinv[...] * (a1[...] + xs[...])
    h1 = jnp.maximum(jnp.dot(pre, w1[...],
                             preferred_element_type=jnp.float32) + b1[...], 0.0)
    y2_ref[...] = jnp.dot(h1, w2[...],
                          preferred_element_type=jnp.float32) * dinv[...]


def _l2_body(a2, y2, dinv, b2, w3, y3_ref):
    h2 = jnp.maximum(dinv[...] * (a2[...] + y2[...]) + b2[...], 0.0)
    y3_ref[...] = jnp.dot(h2, w3[...],
                          preferred_element_type=jnp.float32) * dinv[...]


def _l3_body(a3, y3, dinv, b3, bat, out_ref):
    @pl.when(pl.program_id(0) == 0)
    def _():
        out_ref[...] = jnp.full((GG, 384), -jnp.inf, jnp.float32)

    h3 = jnp.maximum(dinv[...] * (a3[...] + y3[...]) + b3[...], 0.0)
    b = bat[...]
    g0 = jnp.min(b)
    g1 = jnp.minimum(jnp.max(b), 63)

    def body(g, _):
        m = (b == g)
        contrib = jnp.max(jnp.where(m, h3, -jnp.inf), axis=0, keepdims=True)
        cur = out_ref[pl.ds(g, 1), :]
        out_ref[pl.ds(g, 1), :] = jnp.maximum(cur, contrib)
        return 0

    lax.fori_loop(g0, g1 + 1, body, 0)


def _head_body(gm, w1, b1, w2, b2, w3, b3, out_ref):
    g = jnp.maximum(gm[...], 0.0)
    h = jnp.maximum(jnp.dot(g, w1[...],
                            preferred_element_type=jnp.float32) + b1[...], 0.0)
    h = jnp.maximum(jnp.dot(h, w2[...],
                            preferred_element_type=jnp.float32) + b2[...], 0.0)
    lg = jnp.dot(h, w3[...], preferred_element_type=jnp.float32) + b3[...]
    m = jnp.max(lg, axis=0, keepdims=True)
    e = jnp.exp(lg - m)
    out_ref[...] = e / jnp.sum(e, axis=0, keepdims=True)


def _row_spec(w):
    return pl.BlockSpec((RB, w), lambda i: (i, 0))


def _full_spec(shape):
    return pl.BlockSpec(shape, lambda i: tuple(0 for _ in shape))


def _tc_call(body, ins, in_specs, out_shape, out_spec):
    return pl.pallas_call(
        body,
        grid=(98,),
        in_specs=in_specs,
        out_specs=out_spec,
        out_shape=out_shape,
        compiler_params=pltpu.CompilerParams(
            dimension_semantics=("arbitrary",)),
    )(*ins)


def _to_cm(y, C):
    t = y[:NN, :C * 16].reshape(NN, C, 16).transpose(1, 0, 2)
    return jnp.pad(t, ((0, 0), (0, NP16 - NN), (0, 0)))


def _from_cm(a, C, wpad):
    t = a[:, :NN].transpose(1, 0, 2).reshape(NN, C * 16)
    return jnp.pad(t, ((0, NPAD - NN), (0, wpad - C * 16)))


def kernel(x, edge_index, batch, W1, b1, W2, b2, W3, b3,
           L1w, L1b, L2w, L2b, L3w, L3b):
    f32 = jnp.float32
    src, dst = edge_index[0], edge_index[1]
    npad_e = EP - EE
    padi = NN + (jnp.arange(npad_e, dtype=jnp.int32) % 16)
    src2d = jnp.concatenate([src, padi]).reshape(EROWS, 128)
    dst2d = jnp.concatenate([dst, padi]).reshape(EROWS, 128)

    ones_h = jnp.ones((128, 16), f32)
    zeros_h = jnp.zeros((TPR, 16), f32)

    degp = _deg_call(dst2d, ones_h, zeros_h)
    d0 = jnp.pad(degp[0, :NN, 0:1], ((0, NPAD - NN), (0, 0)))
    d1 = jnp.pad(degp[1, :NN, 0:1], ((0, NPAD - NN), (0, 0)))

    xp = jnp.pad(x, ((0, NPAD - NN), (0, 32 - x.shape[1])))
    dinv, xs = _tc_call(
        _prep_body,
        (d0, d1, xp),
        [_row_spec(1), _row_spec(1), _row_spec(32)],
        (jax.ShapeDtypeStruct((NPAD, 1), f32),
         jax.ShapeDtypeStruct((NPAD, 32), f32)),
        (_row_spec(1), _row_spec(32)),
    )

    a1 = _from_cm(_make_agg(2)(_to_cm(xs, 2), src2d, dst2d, zeros_h), 2, 32)

    W1p = jnp.zeros((32, 512), f32).at[:19, :500].set(W1)
    b1p = jnp.zeros((1, 512), f32).at[0, :500].set(b1)
    W2p = jnp.zeros((512, 512), f32).at[:500, :400].set(W2)
    y2 = _tc_call(
        _l1_body,
        (a1, xs, dinv, W1p, b1p, W2p),
        [_row_spec(32), _row_spec(32), _row_spec(1),
         _full_spec((32, 512)), _full_spec((1, 512)), _full_spec((512, 512))],
        jax.ShapeDtypeStruct((NPAD, 512), f32),
        _row_spec(512),
    )

    a2 = _from_cm(_make_agg(25)(_to_cm(y2, 25), src2d, dst2d, zeros_h), 25, 512)

    b2p = jnp.zeros((1, 512), f32).at[0, :400].set(b2)
    W3p = jnp.zeros((512, 384), f32).at[:400, :300].set(W3)
    y3 = _tc_call(
        _l2_body,
        (a2, y2, dinv, b2p, W3p),
        [_row_spec(512), _row_spec(512), _row_spec(1),
         _full_spec((1, 512)), _full_spec((512, 384))],
        jax.ShapeDtypeStruct((NPAD, 384), f32),
        _row_spec(384),
    )

    a3 = _from_cm(_make_agg(19)(_to_cm(y3, 19), src2d, dst2d, zeros_h), 19, 384)

    b3p = jnp.zeros((1, 384), f32).at[0, :300].set(b3)
    batp = jnp.pad(batch, (0, NPAD - NN), constant_values=GG)[:, None]
    gmax = _tc_call(
        _l3_body,
        (a3, y3, dinv, b3p, batp),
        [_row_spec(384), _row_spec(384), _row_spec(1),
         _full_spec((1, 384)), _row_spec(1)],
        jax.ShapeDtypeStruct((GG, 384), f32),
        _full_spec((GG, 384)),
    )

    L1wp = jnp.zeros((384, 256), f32).at[:300, :200].set(L1w)
    L1bp = jnp.zeros((1, 256), f32).at[0, :200].set(L1b)
    L2wp = jnp.zeros((256, 128), f32).at[:200, :100].set(L2w)
    L2bp = jnp.zeros((1, 128), f32).at[0, :100].set(L2b)
    L3wp = jnp.zeros((128, 128), f32).at[:100, :6].set(L3w)
    L3bp = jnp.zeros((1, 128), f32).at[0, :6].set(L3b)

    out = pl.pallas_call(
        _head_body,
        out_shape=jax.ShapeDtypeStruct((GG, 128), f32),
    )(gmax, L1wp, L1bp, L2wp, L2bp, L3wp, L3bp)

    return out[:, :6]


# speedup vs baseline: 3.3769x; 1.1550x over previous
import functools
import jax
import jax.numpy as jnp
from jax import lax
from jax.experimental import pallas as pl
from jax.experimental.pallas import tpu as pltpu
from jax.experimental.pallas import tpu_sc as plsc

NN = 100000
EE = 1600000
GG = 64
RB = 1024
NPAD = 98 * RB
NP16 = NN + 96
EP = 1638400
EROWS = EP // 128
TPR = NP16 // 16

_mesh = plsc.VectorSubcoreMesh(core_axis_name="c", subcore_axis_name="s")


_sc_params = pltpu.CompilerParams(use_tc_tiling_on_sc=False)


@functools.partial(
    pl.kernel, mesh=_mesh, compiler_params=_sc_params,
    out_type=jax.ShapeDtypeStruct((2, NP16, 16), jnp.float32),
    scratch_types=[
        pltpu.VMEM_SHARED((NP16, 16), jnp.float32),
        pltpu.VMEM((16, 128), jnp.int32),
        pltpu.VMEM((128, 16), jnp.float32),
    ],
)
def _deg_call(dst2d, ones_hbm, zeros_hbm, out, acc, dstv, onesv):
    cid = lax.axis_index("c")
    sid = lax.axis_index("s")
    pltpu.sync_copy(ones_hbm, onesv)
    pltpu.sync_copy(zeros_hbm, acc.at[pl.ds(sid * TPR, TPR)])
    plsc.subcore_barrier()
    wid = cid * 16 + sid
    base = wid * (EROWS // 32)

    def w(i, _):
        pltpu.sync_copy(dst2d.at[pl.ds(base + i * 16, 16)], dstv)
        for j in range(16):
            pltpu.sync_copy(onesv, acc.at[dstv.at[j]], add=True)
        return 0

    lax.fori_loop(0, 25, w, 0)
    plsc.subcore_barrier()
    pltpu.sync_copy(acc.at[pl.ds(sid * TPR, TPR)],
                    out.at[cid].at[pl.ds(sid * TPR, TPR)])


def _make_agg(C):
    @functools.partial(
        pl.kernel, mesh=_mesh, compiler_params=_sc_params,
        out_type=jax.ShapeDtypeStruct((C, NP16, 16), jnp.float32),
        scratch_types=[
            pltpu.VMEM_SHARED((NP16, 16), jnp.float32),
            pltpu.VMEM((4, 128), jnp.int32),
            pltpu.VMEM((4, 128), jnp.int32),
            pltpu.VMEM((4, 128), jnp.int32),
            pltpu.VMEM((4, 128), jnp.int32),
            pltpu.VMEM((512, 16), jnp.float32),
            pltpu.VMEM((512, 16), jnp.float32),
            pltpu.SemaphoreType.DMA,
            pltpu.SemaphoreType.DMA,
        ],
    )
    def _agg(ycm, src2d, dst2d, zeros_hbm, out,
             acc, sv0, dv0, sv1, dv1, r0, r1, sg0, sg1):
        cid = lax.axis_index("c")
        sid = lax.axis_index("s")
        nsl = pl.ds(sid * TPR, TPR)
        base = sid * (EROWS // 16)
        NW = (EROWS // 16) // 4

        def chunk(k, _):
            @pl.when((k % 2) == cid)
            def _():
                pltpu.sync_copy(zeros_hbm, acc.at[nsl])
                plsc.subcore_barrier()

                def lidx(w, sv, dv):
                    off = base + w * 4
                    pltpu.sync_copy(src2d.at[pl.ds(off, 4)], sv)
                    pltpu.sync_copy(dst2d.at[pl.ds(off, 4)], dv)

                def fire(sv, r, sg):
                    for j in range(4):
                        pltpu.async_copy(ycm.at[k].at[sv.at[j]],
                                         r.at[pl.ds(j * 128, 128)], sg)

                def drain(sv, r, sg):
                    for j in range(4):
                        pltpu.make_async_copy(ycm.at[k].at[sv.at[j]],
                                              r.at[pl.ds(j * 128, 128)],
                                              sg).wait()

                def scat(dv, r):
                    for j in range(4):
                        pltpu.sync_copy(r.at[pl.ds(j * 128, 128)],
                                        acc.at[dv.at[j]], add=True)

                lidx(0, sv0, dv0)
                fire(sv0, r0, sg0)
                lidx(1, sv1, dv1)

                def w(i, _):
                    fire(sv1, r1, sg1)
                    drain(sv0, r0, sg0)
                    scat(dv0, r0)

                    @pl.when(i < NW // 2 - 1)
                    def _():
                        lidx(2 * i + 2, sv0, dv0)
                        fire(sv0, r0, sg0)

                    drain(sv1, r1, sg1)
                    scat(dv1, r1)

                    @pl.when(i < NW // 2 - 1)
                    def _():
                        lidx(2 * i + 3, sv1, dv1)
                    return 0

                lax.fori_loop(0, NW // 2, w, 0)
                plsc.subcore_barrier()
                pltpu.sync_copy(acc.at[nsl], out.at[k].at[nsl])
            return 0

        lax.fori_loop(0, C, chunk, 0)

    return _agg


def _prep_body(d0, d1, xp, dinv_ref, xs_ref):
    deg = d0[...] + d1[...] + 1.0
    dv = lax.rsqrt(deg)
    dinv_ref[...] = dv
    xs_ref[...] = xp[...] * dv


def _l1_body(a1, xs, dinv, w1, b1, w2, y2_ref):
    pre = dinv[...] * (a1[...] + xs[...])
    h1 = jnp.maximum(jnp.dot(pre, w1[...],
                             preferred_element_type=jnp.float32) + b1[...], 0.0)
    y2_ref[...] = jnp.dot(h1, w2[...],
                          preferred_element_type=jnp.float32) * dinv[...]


def _l2_body(a2, y2, dinv, b2, w3, y3_ref):
    h2 = jnp.maximum(dinv[...] * (a2[...] + y2[...]) + b2[...], 0.0)
    y3_ref[...] = jnp.dot(h2, w3[...],
                          preferred_element_type=jnp.float32) * dinv[...]


def _l3_body(a3, y3, dinv, b3, bat, out_ref):
    @pl.when(pl.program_id(0) == 0)
    def _():
        out_ref[...] = jnp.full((GG, 384), -jnp.inf, jnp.float32)

    h3 = jnp.maximum(dinv[...] * (a3[...] + y3[...]) + b3[...], 0.0)
    b = bat[...]
    g0 = jnp.min(b)
    g1 = jnp.minimum(jnp.max(b), 63)

    def body(g, _):
        m = (b == g)
        contrib = jnp.max(jnp.where(m, h3, -jnp.inf), axis=0, keepdims=True)
        cur = out_ref[pl.ds(g, 1), :]
        out_ref[pl.ds(g, 1), :] = jnp.maximum(cur, contrib)
        return 0

    lax.fori_loop(g0, g1 + 1, body, 0)


def _head_body(gm, w1, b1, w2, b2, w3, b3, out_ref):
    g = jnp.maximum(gm[...], 0.0)
    h = jnp.maximum(jnp.dot(g, w1[...],
                            preferred_element_type=jnp.float32) + b1[...], 0.0)
    h = jnp.maximum(jnp.dot(h, w2[...],
                            preferred_element_type=jnp.float32) + b2[...], 0.0)
    lg = jnp.dot(h, w3[...], preferred_element_type=jnp.float32) + b3[...]
    m = jnp.max(lg, axis=0, keepdims=True)
    e = jnp.exp(lg - m)
    out_ref[...] = e / jnp.sum(e, axis=0, keepdims=True)


def _row_spec(w):
    return pl.BlockSpec((RB, w), lambda i: (i, 0))


def _full_spec(shape):
    return pl.BlockSpec(shape, lambda i: tuple(0 for _ in shape))


def _tc_call(body, ins, in_specs, out_shape, out_spec):
    return pl.pallas_call(
        body,
        grid=(98,),
        in_specs=in_specs,
        out_specs=out_spec,
        out_shape=out_shape,
        compiler_params=pltpu.CompilerParams(
            dimension_semantics=("arbitrary",)),
    )(*ins)


def _to_cm(y, C):
    t = y[:NN, :C * 16].reshape(NN, C, 16).transpose(1, 0, 2)
    return jnp.pad(t, ((0, 0), (0, NP16 - NN), (0, 0)))


def _from_cm(a, C, wpad):
    t = a[:, :NN].transpose(1, 0, 2).reshape(NN, C * 16)
    return jnp.pad(t, ((0, NPAD - NN), (0, wpad - C * 16)))


def kernel(x, edge_index, batch, W1, b1, W2, b2, W3, b3,
           L1w, L1b, L2w, L2b, L3w, L3b):
    f32 = jnp.float32
    src, dst = edge_index[0], edge_index[1]
    npad_e = EP - EE
    padi = NN + (jnp.arange(npad_e, dtype=jnp.int32) % 16)
    src2d = jnp.concatenate([src, padi]).reshape(EROWS, 128)
    dst2d = jnp.concatenate([dst, padi]).reshape(EROWS, 128)

    ones_h = jnp.ones((128, 16), f32)
    zeros_h = jnp.zeros((TPR, 16), f32)

    degp = _deg_call(dst2d, ones_h, zeros_h)
    d0 = jnp.pad(degp[0, :NN, 0:1], ((0, NPAD - NN), (0, 0)))
    d1 = jnp.pad(degp[1, :NN, 0:1], ((0, NPAD - NN), (0, 0)))

    xp = jnp.pad(x, ((0, NPAD - NN), (0, 32 - x.shape[1])))
    dinv, xs = _tc_call(
        _prep_body,
        (d0, d1, xp),
        [_row_spec(1), _row_spec(1), _row_spec(32)],
        (jax.ShapeDtypeStruct((NPAD, 1), f32),
         jax.ShapeDtypeStruct((NPAD, 32), f32)),
        (_row_spec(1), _row_spec(32)),
    )

    a1 = _from_cm(_make_agg(2)(_to_cm(xs, 2), src2d, dst2d, zeros_h), 2, 32)

    W1p = jnp.zeros((32, 512), f32).at[:19, :500].set(W1)
    b1p = jnp.zeros((1, 512), f32).at[0, :500].set(b1)
    W2p = jnp.zeros((512, 512), f32).at[:500, :400].set(W2)
    y2 = _tc_call(
        _l1_body,
        (a1, xs, dinv, W1p, b1p, W2p),
        [_row_spec(32), _row_spec(32), _row_spec(1),
         _full_spec((32, 512)), _full_spec((1, 512)), _full_spec((512, 512))],
        jax.ShapeDtypeStruct((NPAD, 512), f32),
        _row_spec(512),
    )

    a2 = _from_cm(_make_agg(25)(_to_cm(y2, 25), src2d, dst2d, zeros_h), 25, 512)

    b2p = jnp.zeros((1, 512), f32).at[0, :400].set(b2)
    W3p = jnp.zeros((512, 384), f32).at[:400, :300].set(W3)
    y3 = _tc_call(
        _l2_body,
        (a2, y2, dinv, b2p, W3p),
        [_row_spec(512), _row_spec(512), _row_spec(1),
         _full_spec((1, 512)), _full_spec((512, 384))],
        jax.ShapeDtypeStruct((NPAD, 384), f32),
        _row_spec(384),
    )

    a3 = _from_cm(_make_agg(19)(_to_cm(y3, 19), src2d, dst2d, zeros_h), 19, 384)

    b3p = jnp.zeros((1, 384), f32).at[0, :300].set(b3)
    batp = jnp.pad(batch, (0, NPAD - NN), constant_values=GG)[:, None]
    gmax = _tc_call(
        _l3_body,
        (a3, y3, dinv, b3p, batp),
        [_row_spec(384), _row_spec(384), _row_spec(1),
         _full_spec((1, 384)), _row_spec(1)],
        jax.ShapeDtypeStruct((GG, 384), f32),
        _full_spec((GG, 384)),
    )

    L1wp = jnp.zeros((384, 256), f32).at[:300, :200].set(L1w)
    L1bp = jnp.zeros((1, 256), f32).at[0, :200].set(L1b)
    L2wp = jnp.zeros((256, 128), f32).at[:200, :100].set(L2w)
    L2bp = jnp.zeros((1, 128), f32).at[0, :100].set(L2b)
    L3wp = jnp.zeros((128, 128), f32).at[:100, :6].set(L3w)
    L3bp = jnp.zeros((1, 128), f32).at[0, :6].set(L3b)

    out = pl.pallas_call(
        _head_body,
        out_shape=jax.ShapeDtypeStruct((GG, 128), f32),
    )(gmax, L1wp, L1bp, L2wp, L2bp, L3wp, L3bp)

    return out[:, :6]


# speedup vs baseline: 4.2078x; 1.2461x over previous
import functools
import jax
import jax.numpy as jnp
from jax import lax
from jax.experimental import pallas as pl
from jax.experimental.pallas import tpu as pltpu
from jax.experimental.pallas import tpu_sc as plsc

NN = 100000
EE = 1600000
GG = 64
RB = 1024
NPAD = 98 * RB
NP16 = NN + 96
EP = 1638400
EROWS = EP // 128
TPR = NP16 // 16

_mesh = plsc.VectorSubcoreMesh(core_axis_name="c", subcore_axis_name="s")


_sc_params = pltpu.CompilerParams(use_tc_tiling_on_sc=False)


@functools.partial(
    pl.kernel, mesh=_mesh, compiler_params=_sc_params,
    out_type=jax.ShapeDtypeStruct((2, NP16, 16), jnp.float32),
    scratch_types=[
        pltpu.VMEM_SHARED((NP16, 16), jnp.float32),
        pltpu.VMEM((16, 128), jnp.int32),
        pltpu.VMEM((128, 16), jnp.float32),
    ],
)
def _deg_call(dst2d, ones_hbm, zeros_hbm, out, acc, dstv, onesv):
    cid = lax.axis_index("c")
    sid = lax.axis_index("s")
    pltpu.sync_copy(ones_hbm, onesv)
    pltpu.sync_copy(zeros_hbm, acc.at[pl.ds(sid * TPR, TPR)])
    plsc.subcore_barrier()
    wid = cid * 16 + sid
    base = wid * (EROWS // 32)

    def w(i, _):
        pltpu.sync_copy(dst2d.at[pl.ds(base + i * 16, 16)], dstv)
        for j in range(16):
            pltpu.sync_copy(onesv, acc.at[dstv.at[j]], add=True)
        return 0

    lax.fori_loop(0, 25, w, 0)
    plsc.subcore_barrier()
    pltpu.sync_copy(acc.at[pl.ds(sid * TPR, TPR)],
                    out.at[cid].at[pl.ds(sid * TPR, TPR)])


def _make_agg(C):
    @functools.partial(
        pl.kernel, mesh=_mesh, compiler_params=_sc_params,
        out_type=jax.ShapeDtypeStruct((NP16, C * 16), jnp.float32),
        scratch_types=[
            pltpu.VMEM_SHARED((NP16, 16), jnp.float32),
            pltpu.VMEM((4, 128), jnp.int32),
            pltpu.VMEM((4, 128), jnp.int32),
            pltpu.VMEM((4, 128), jnp.int32),
            pltpu.VMEM((4, 128), jnp.int32),
            pltpu.VMEM((512, 16), jnp.float32),
            pltpu.VMEM((512, 16), jnp.float32),
            pltpu.SemaphoreType.DMA,
            pltpu.SemaphoreType.DMA,
        ],
    )
    def _agg(ycm, src2d, dst2d, zeros_hbm, out,
             acc, sv0, dv0, sv1, dv1, r0, r1, sg0, sg1):
        cid = lax.axis_index("c")
        sid = lax.axis_index("s")
        nsl = pl.ds(sid * TPR, TPR)
        base = sid * (EROWS // 16)
        NW = (EROWS // 16) // 4

        def chunk(k, _):
            @pl.when((k % 2) == cid)
            def _():
                pltpu.sync_copy(zeros_hbm, acc.at[nsl])
                plsc.subcore_barrier()

                def lidx(w, sv, dv):
                    off = base + w * 4
                    pltpu.sync_copy(src2d.at[pl.ds(off, 4)], sv)
                    pltpu.sync_copy(dst2d.at[pl.ds(off, 4)], dv)

                def fire(sv, r, sg):
                    for j in range(4):
                        pltpu.async_copy(ycm.at[k].at[sv.at[j]],
                                         r.at[pl.ds(j * 128, 128)], sg)

                def drain(sv, r, sg):
                    for j in range(4):
                        pltpu.make_async_copy(ycm.at[k].at[sv.at[j]],
                                              r.at[pl.ds(j * 128, 128)],
                                              sg).wait()

                def scat(dv, r):
                    for j in range(4):
                        pltpu.sync_copy(r.at[pl.ds(j * 128, 128)],
                                        acc.at[dv.at[j]], add=True)

                lidx(0, sv0, dv0)
                fire(sv0, r0, sg0)
                lidx(1, sv1, dv1)

                def w(i, _):
                    fire(sv1, r1, sg1)
                    drain(sv0, r0, sg0)
                    scat(dv0, r0)

                    @pl.when(i < NW // 2 - 1)
                    def _():
                        lidx(2 * i + 2, sv0, dv0)
                        fire(sv0, r0, sg0)

                    drain(sv1, r1, sg1)
                    scat(dv1, r1)

                    @pl.when(i < NW // 2 - 1)
                    def _():
                        lidx(2 * i + 3, sv1, dv1)
                    return 0

                lax.fori_loop(0, NW // 2, w, 0)
                plsc.subcore_barrier()
                pltpu.sync_copy(acc.at[nsl],
                                out.at[nsl, pl.ds(k * 16, 16)])
            return 0

        lax.fori_loop(0, C, chunk, 0)

    return _agg


def _prep_body(d0, d1, xp, dinv_ref, xs_ref):
    deg = d0[...] + d1[...] + 1.0
    dv = lax.rsqrt(deg)
    dinv_ref[...] = dv
    xs_ref[...] = xp[...] * dv


def _l1_body(a1, xs, dinv, w1, b1, w2, y2_ref):
    pre = dinv[...] * (a1[...] + xs[...])
    h1 = jnp.maximum(jnp.dot(pre, w1[...],
                             preferred_element_type=jnp.float32) + b1[...], 0.0)
    y2_ref[...] = jnp.dot(h1, w2[...],
                          preferred_element_type=jnp.float32) * dinv[...]


def _l2_body(a2, y2, dinv, b2, w3, y3_ref):
    h2 = jnp.maximum(dinv[...] * (a2[...] + y2[...]) + b2[...], 0.0)
    y3_ref[...] = jnp.dot(h2, w3[...],
                          preferred_element_type=jnp.float32) * dinv[...]


def _l3_body(a3, y3, dinv, b3, bat, out_ref):
    @pl.when(pl.program_id(0) == 0)
    def _():
        out_ref[...] = jnp.full((GG, 384), -jnp.inf, jnp.float32)

    h3 = jnp.maximum(dinv[...] * (a3[...] + y3[...]) + b3[...], 0.0)
    b = bat[...]
    g0 = jnp.min(b)
    g1 = jnp.minimum(jnp.max(b), 63)

    def body(g, _):
        m = (b == g)
        contrib = jnp.max(jnp.where(m, h3, -jnp.inf), axis=0, keepdims=True)
        cur = out_ref[pl.ds(g, 1), :]
        out_ref[pl.ds(g, 1), :] = jnp.maximum(cur, contrib)
        return 0

    lax.fori_loop(g0, g1 + 1, body, 0)


def _head_body(gm, w1, b1, w2, b2, w3, b3, out_ref):
    g = jnp.maximum(gm[...], 0.0)
    h = jnp.maximum(jnp.dot(g, w1[...],
                            preferred_element_type=jnp.float32) + b1[...], 0.0)
    h = jnp.maximum(jnp.dot(h, w2[...],
                            preferred_element_type=jnp.float32) + b2[...], 0.0)
    lg = jnp.dot(h, w3[...], preferred_element_type=jnp.float32) + b3[...]
    m = jnp.max(lg, axis=0, keepdims=True)
    e = jnp.exp(lg - m)
    out_ref[...] = e / jnp.sum(e, axis=0, keepdims=True)


def _row_spec(w):
    return pl.BlockSpec((RB, w), lambda i: (i, 0))


def _full_spec(shape):
    return pl.BlockSpec(shape, lambda i: tuple(0 for _ in shape))


def _tc_call(body, ins, in_specs, out_shape, out_spec):
    return pl.pallas_call(
        body,
        grid=(98,),
        in_specs=in_specs,
        out_specs=out_spec,
        out_shape=out_shape,
        compiler_params=pltpu.CompilerParams(
            dimension_semantics=("arbitrary",)),
    )(*ins)


def _to_cm(y, C):
    t = y[:NN, :C * 16].reshape(NN, C, 16).transpose(1, 0, 2)
    return jnp.pad(t, ((0, 0), (0, NP16 - NN), (0, 0)))


def _rpad(a, wpad):
    return jnp.pad(a, ((0, NPAD - NP16), (0, wpad - a.shape[1])))


def kernel(x, edge_index, batch, W1, b1, W2, b2, W3, b3,
           L1w, L1b, L2w, L2b, L3w, L3b):
    f32 = jnp.float32
    src, dst = edge_index[0], edge_index[1]
    npad_e = EP - EE
    padi = NN + (jnp.arange(npad_e, dtype=jnp.int32) % 16)
    src2d = jnp.concatenate([src, padi]).reshape(EROWS, 128)
    dst2d = jnp.concatenate([dst, padi]).reshape(EROWS, 128)

    ones_h = jnp.ones((128, 16), f32)
    zeros_h = jnp.zeros((TPR, 16), f32)

    degp = _deg_call(dst2d, ones_h, zeros_h)
    d0 = jnp.pad(degp[0, :NN, 0:1], ((0, NPAD - NN), (0, 0)))
    d1 = jnp.pad(degp[1, :NN, 0:1], ((0, NPAD - NN), (0, 0)))

    xp = jnp.pad(x, ((0, NPAD - NN), (0, 32 - x.shape[1])))
    dinv, xs = _tc_call(
        _prep_body,
        (d0, d1, xp),
        [_row_spec(1), _row_spec(1), _row_spec(32)],
        (jax.ShapeDtypeStruct((NPAD, 1), f32),
         jax.ShapeDtypeStruct((NPAD, 32), f32)),
        (_row_spec(1), _row_spec(32)),
    )

    a1 = _rpad(_make_agg(2)(_to_cm(xs, 2), src2d, dst2d, zeros_h), 32)

    W1p = jnp.zeros((32, 512), f32).at[:19, :500].set(W1)
    b1p = jnp.zeros((1, 512), f32).at[0, :500].set(b1)
    W2p = jnp.zeros((512, 512), f32).at[:500, :400].set(W2)
    y2 = _tc_call(
        _l1_body,
        (a1, xs, dinv, W1p, b1p, W2p),
        [_row_spec(32), _row_spec(32), _row_spec(1),
         _full_spec((32, 512)), _full_spec((1, 512)), _full_spec((512, 512))],
        jax.ShapeDtypeStruct((NPAD, 512), f32),
        _row_spec(512),
    )

    a2 = _rpad(_make_agg(25)(_to_cm(y2, 25), src2d, dst2d, zeros_h), 512)

    b2p = jnp.zeros((1, 512), f32).at[0, :400].set(b2)
    W3p = jnp.zeros((512, 384), f32).at[:400, :300].set(W3)
    y3 = _tc_call(
        _l2_body,
        (a2, y2, dinv, b2p, W3p),
        [_row_spec(512), _row_spec(512), _row_spec(1),
         _full_spec((1, 512)), _full_spec((512, 384))],
        jax.ShapeDtypeStruct((NPAD, 384), f32),
        _row_spec(384),
    )

    a3 = _rpad(_make_agg(19)(_to_cm(y3, 19), src2d, dst2d, zeros_h), 384)

    b3p = jnp.zeros((1, 384), f32).at[0, :300].set(b3)
    batp = jnp.pad(batch, (0, NPAD - NN), constant_values=GG)[:, None]
    gmax = _tc_call(
        _l3_body,
        (a3, y3, dinv, b3p, batp),
        [_row_spec(384), _row_spec(384), _row_spec(1),
         _full_spec((1, 384)), _row_spec(1)],
        jax.ShapeDtypeStruct((GG, 384), f32),
        _full_spec((GG, 384)),
    )

    L1wp = jnp.zeros((384, 256), f32).at[:300, :200].set(L1w)
    L1bp = jnp.zeros((1, 256), f32).at[0, :200].set(L1b)
    L2wp = jnp.zeros((256, 128), f32).at[:200, :100].set(L2w)
    L2bp = jnp.zeros((1, 128), f32).at[0, :100].set(L2b)
    L3wp = jnp.zeros((128, 128), f32).at[:100, :6].set(L3w)
    L3bp = jnp.zeros((1, 128), f32).at[0, :6].set(L3b)

    out = pl.pallas_call(
        _head_body,
        out_shape=jax.ShapeDtypeStruct((GG, 128), f32),
    )(gmax, L1wp, L1bp, L2wp, L2bp, L3wp, L3bp)

    return out[:, :6]


# speedup vs baseline: 4.5699x; 1.0861x over previous
import functools
import jax
import jax.numpy as jnp
from jax import lax
from jax.experimental import pallas as pl
from jax.experimental.pallas import tpu as pltpu
from jax.experimental.pallas import tpu_sc as plsc

NN = 100000
EE = 1600000
GG = 64
RB = 1024
NPAD = 98 * RB
NP16 = NN + 96
EP = 1638400
EROWS = EP // 128
TPR = NP16 // 16

_mesh = plsc.VectorSubcoreMesh(core_axis_name="c", subcore_axis_name="s")


_sc_params = pltpu.CompilerParams(use_tc_tiling_on_sc=False)


@functools.partial(
    pl.kernel, mesh=_mesh, compiler_params=_sc_params,
    out_type=jax.ShapeDtypeStruct((2, NP16, 16), jnp.float32),
    scratch_types=[
        pltpu.VMEM_SHARED((NP16, 16), jnp.float32),
        pltpu.VMEM((16, 128), jnp.int32),
        pltpu.VMEM((128, 16), jnp.float32),
    ],
)
def _deg_call(dst2d, ones_hbm, zeros_hbm, out, acc, dstv, onesv):
    cid = lax.axis_index("c")
    sid = lax.axis_index("s")
    pltpu.sync_copy(ones_hbm, onesv)
    pltpu.sync_copy(zeros_hbm, acc.at[pl.ds(sid * TPR, TPR)])
    plsc.subcore_barrier()
    wid = cid * 16 + sid
    base = wid * (EROWS // 32)

    def w(i, _):
        pltpu.sync_copy(dst2d.at[pl.ds(base + i * 16, 16)], dstv)
        for j in range(16):
            pltpu.sync_copy(onesv, acc.at[dstv.at[j]], add=True)
        return 0

    lax.fori_loop(0, 25, w, 0)
    plsc.subcore_barrier()
    pltpu.sync_copy(acc.at[pl.ds(sid * TPR, TPR)],
                    out.at[cid].at[pl.ds(sid * TPR, TPR)])


def _make_agg(C):
    @functools.partial(
        pl.kernel, mesh=_mesh, compiler_params=_sc_params,
        out_type=jax.ShapeDtypeStruct((NP16, C * 16), jnp.float32),
        scratch_types=[
            pltpu.VMEM_SHARED((NP16, 16), jnp.float32),
            pltpu.VMEM((3, 4, 128), jnp.int32),
            pltpu.VMEM((3, 4, 128), jnp.int32),
            pltpu.VMEM((3, 512, 16), jnp.float32),
            pltpu.SemaphoreType.DMA,
            pltpu.SemaphoreType.DMA,
            pltpu.SemaphoreType.DMA,
            pltpu.SemaphoreType.DMA,
            pltpu.SemaphoreType.DMA,
            pltpu.SemaphoreType.DMA,
        ],
    )
    def _agg(ycm, src2d, dst2d, zeros_hbm, out,
             acc, svb, dvb, rb, sg0, sg1, sg2, ss0, ss1, ss2):
        cid = lax.axis_index("c")
        sid = lax.axis_index("s")
        nsl = pl.ds(sid * TPR, TPR)
        base = sid * (EROWS // 16)
        NW = (EROWS // 16) // 4
        sg = [sg0, sg1, sg2]
        ss = [ss0, ss1, ss2]

        def chunk(k, _):
            @pl.when((k % 2) == cid)
            def _():
                pltpu.sync_copy(zeros_hbm, acc.at[nsl])
                plsc.subcore_barrier()

                def lidx(w, b):
                    off = base + w * 4
                    pltpu.sync_copy(src2d.at[pl.ds(off, 4)], svb.at[b])
                    pltpu.sync_copy(dst2d.at[pl.ds(off, 4)], dvb.at[b])

                def fire_g(b):
                    for j in range(4):
                        pltpu.async_copy(ycm.at[k].at[svb.at[b].at[j]],
                                         rb.at[b].at[pl.ds(j * 128, 128)],
                                         sg[b])

                def drain_g(b):
                    for j in range(4):
                        pltpu.make_async_copy(
                            ycm.at[k].at[svb.at[b].at[j]],
                            rb.at[b].at[pl.ds(j * 128, 128)], sg[b]).wait()

                def fire_s(b):
                    for j in range(4):
                        pltpu.async_copy(rb.at[b].at[pl.ds(j * 128, 128)],
                                         acc.at[dvb.at[b].at[j]], ss[b],
                                         add=True)

                def drain_s(b):
                    for j in range(4):
                        pltpu.make_async_copy(
                            rb.at[b].at[pl.ds(j * 128, 128)],
                            acc.at[dvb.at[b].at[j]], ss[b]).wait()

                def step(w, u, guard_first):
                    a, n = u % 3, (u + 1) % 3
                    if guard_first:
                        @pl.when(w >= 2)
                        def _():
                            drain_s(n)
                    else:
                        drain_s(n)
                    lidx(w + 1, n)
                    fire_g(n)
                    drain_g(a)
                    fire_s(a)

                lidx(0, 0)
                fire_g(0)

                def body(i, _):
                    w = 3 * i
                    step(w, 0, True)
                    step(w + 1, 1, True)
                    step(w + 2, 2, False)
                    return 0

                lax.fori_loop(0, (NW - 2) // 3, body, 0)
                drain_s(1)
                lidx(NW - 1, 1)
                fire_g(1)
                drain_g(0)
                fire_s(0)
                drain_s(2)
                drain_g(1)
                fire_s(1)
                drain_s(0)
                drain_s(1)
                plsc.subcore_barrier()
                pltpu.sync_copy(acc.at[nsl],
                                out.at[nsl, pl.ds(k * 16, 16)])
            return 0

        lax.fori_loop(0, C, chunk, 0)

    return _agg


def _prep_body(d0, d1, xp, dinv_ref, xs_ref):
    deg = d0[...] + d1[...] + 1.0
    dv = lax.rsqrt(deg)
    dinv_ref[...] = dv
    xs_ref[...] = xp[...] * dv


def _l1_body(a1, xs, dinv, w1, b1, w2, y2_ref):
    pre = dinv[...] * (a1[...] + xs[...])
    h1 = jnp.maximum(jnp.dot(pre, w1[...],
                             preferred_element_type=jnp.float32) + b1[...], 0.0)
    y2_ref[...] = jnp.dot(h1, w2[...],
                          preferred_element_type=jnp.float32) * dinv[...]


def _l2_body(a2, y2, dinv, b2, w3, y3_ref):
    h2 = jnp.maximum(dinv[...] * (a2[...] + y2[...]) + b2[...], 0.0)
    y3_ref[...] = jnp.dot(h2, w3[...],
                          preferred_element_type=jnp.float32) * dinv[...]


def _l3_body(a3, y3, dinv, b3, bat, out_ref):
    @pl.when(pl.program_id(0) == 0)
    def _():
        out_ref[...] = jnp.full((GG, 384), -jnp.inf, jnp.float32)

    h3 = jnp.maximum(dinv[...] * (a3[...] + y3[...]) + b3[...], 0.0)
    b = bat[...]
    g0 = jnp.min(b)
    g1 = jnp.minimum(jnp.max(b), 63)

    def body(g, _):
        m = (b == g)
        contrib = jnp.max(jnp.where(m, h3, -jnp.inf), axis=0, keepdims=True)
        cur = out_ref[pl.ds(g, 1), :]
        out_ref[pl.ds(g, 1), :] = jnp.maximum(cur, contrib)
        return 0

    lax.fori_loop(g0, g1 + 1, body, 0)


def _head_body(gm, w1, b1, w2, b2, w3, b3, out_ref):
    g = jnp.maximum(gm[...], 0.0)
    h = jnp.maximum(jnp.dot(g, w1[...],
                            preferred_element_type=jnp.float32) + b1[...], 0.0)
    h = jnp.maximum(jnp.dot(h, w2[...],
                            preferred_element_type=jnp.float32) + b2[...], 0.0)
    lg = jnp.dot(h, w3[...], preferred_element_type=jnp.float32) + b3[...]
    m = jnp.max(lg, axis=0, keepdims=True)
    e = jnp.exp(lg - m)
    out_ref[...] = e / jnp.sum(e, axis=0, keepdims=True)


def _row_spec(w):
    return pl.BlockSpec((RB, w), lambda i: (i, 0))


def _full_spec(shape):
    return pl.BlockSpec(shape, lambda i: tuple(0 for _ in shape))


def _tc_call(body, ins, in_specs, out_shape, out_spec):
    return pl.pallas_call(
        body,
        grid=(98,),
        in_specs=in_specs,
        out_specs=out_spec,
        out_shape=out_shape,
        compiler_params=pltpu.CompilerParams(
            dimension_semantics=("arbitrary",)),
    )(*ins)


def _to_cm(y, C):
    t = y[:NN, :C * 16].reshape(NN, C, 16).transpose(1, 0, 2)
    return jnp.pad(t, ((0, 0), (0, NP16 - NN), (0, 0)))


def _rpad(a, wpad):
    return jnp.pad(a, ((0, NPAD - NP16), (0, wpad - a.shape[1])))


def kernel(x, edge_index, batch, W1, b1, W2, b2, W3, b3,
           L1w, L1b, L2w, L2b, L3w, L3b):
    f32 = jnp.float32
    src, dst = edge_index[0], edge_index[1]
    npad_e = EP - EE
    padi = NN + (jnp.arange(npad_e, dtype=jnp.int32) % 16)
    src2d = jnp.concatenate([src, padi]).reshape(EROWS, 128)
    dst2d = jnp.concatenate([dst, padi]).reshape(EROWS, 128)

    ones_h = jnp.ones((128, 16), f32)
    zeros_h = jnp.zeros((TPR, 16), f32)

    degp = _deg_call(dst2d, ones_h, zeros_h)
    d0 = jnp.pad(degp[0, :NN, 0:1], ((0, NPAD - NN), (0, 0)))
    d1 = jnp.pad(degp[1, :NN, 0:1], ((0, NPAD - NN), (0, 0)))

    xp = jnp.pad(x, ((0, NPAD - NN), (0, 32 - x.shape[1])))
    dinv, xs = _tc_call(
        _prep_body,
        (d0, d1, xp),
        [_row_spec(1), _row_spec(1), _row_spec(32)],
        (jax.ShapeDtypeStruct((NPAD, 1), f32),
         jax.ShapeDtypeStruct((NPAD, 32), f32)),
        (_row_spec(1), _row_spec(32)),
    )

    a1 = _rpad(_make_agg(2)(_to_cm(xs, 2), src2d, dst2d, zeros_h), 32)

    W1p = jnp.zeros((32, 512), f32).at[:19, :500].set(W1)
    b1p = jnp.zeros((1, 512), f32).at[0, :500].set(b1)
    W2p = jnp.zeros((512, 512), f32).at[:500, :400].set(W2)
    y2 = _tc_call(
        _l1_body,
        (a1, xs, dinv, W1p, b1p, W2p),
        [_row_spec(32), _row_spec(32), _row_spec(1),
         _full_spec((32, 512)), _full_spec((1, 512)), _full_spec((512, 512))],
        jax.ShapeDtypeStruct((NPAD, 512), f32),
        _row_spec(512),
    )

    a2 = _rpad(_make_agg(25)(_to_cm(y2, 25), src2d, dst2d, zeros_h), 512)

    b2p = jnp.zeros((1, 512), f32).at[0, :400].set(b2)
    W3p = jnp.zeros((512, 384), f32).at[:400, :300].set(W3)
    y3 = _tc_call(
        _l2_body,
        (a2, y2, dinv, b2p, W3p),
        [_row_spec(512), _row_spec(512), _row_spec(1),
         _full_spec((1, 512)), _full_spec((512, 384))],
        jax.ShapeDtypeStruct((NPAD, 384), f32),
        _row_spec(384),
    )

    a3 = _rpad(_make_agg(19)(_to_cm(y3, 19), src2d, dst2d, zeros_h), 384)

    b3p = jnp.zeros((1, 384), f32).at[0, :300].set(b3)
    batp = jnp.pad(batch, (0, NPAD - NN), constant_values=GG)[:, None]
    gmax = _tc_call(
        _l3_body,
        (a3, y3, dinv, b3p, batp),
        [_row_spec(384), _row_spec(384), _row_spec(1),
         _full_spec((1, 384)), _row_spec(1)],
        jax.ShapeDtypeStruct((GG, 384), f32),
        _full_spec((GG, 384)),
    )

    L1wp = jnp.zeros((384, 256), f32).at[:300, :200].set(L1w)
    L1bp = jnp.zeros((1, 256), f32).at[0, :200].set(L1b)
    L2wp = jnp.zeros((256, 128), f32).at[:200, :100].set(L2w)
    L2bp = jnp.zeros((1, 128), f32).at[0, :100].set(L2b)
    L3wp = jnp.zeros((128, 128), f32).at[:100, :6].set(L3w)
    L3bp = jnp.zeros((1, 128), f32).at[0, :6].set(L3b)

    out = pl.pallas_call(
        _head_body,
        out_shape=jax.ShapeDtypeStruct((GG, 128), f32),
    )(gmax, L1wp, L1bp, L2wp, L2bp, L3wp, L3bp)

    return out[:, :6]


# speedup vs baseline: 4.6428x; 1.0160x over previous
import functools
import jax
import jax.numpy as jnp
from jax import lax
from jax.experimental import pallas as pl
from jax.experimental.pallas import tpu as pltpu
from jax.experimental.pallas import tpu_sc as plsc

NN = 100000
EE = 1600000
GG = 64
RB = 1024
NPAD = 98 * RB
NP16 = NN + 96
EP = 1638400
EROWS = EP // 128
TPR = NP16 // 16

_mesh = plsc.VectorSubcoreMesh(core_axis_name="c", subcore_axis_name="s")


_sc_params = pltpu.CompilerParams(use_tc_tiling_on_sc=False)


@functools.partial(
    pl.kernel, mesh=_mesh, compiler_params=_sc_params,
    out_type=jax.ShapeDtypeStruct((2, NP16, 16), jnp.float32),
    scratch_types=[
        pltpu.VMEM_SHARED((NP16, 16), jnp.float32),
        pltpu.VMEM((16, 128), jnp.int32),
        pltpu.VMEM((128, 16), jnp.float32),
    ],
)
def _deg_call(dst2d, ones_hbm, zeros_hbm, out, acc, dstv, onesv):
    cid = lax.axis_index("c")
    sid = lax.axis_index("s")
    pltpu.sync_copy(ones_hbm, onesv)
    pltpu.sync_copy(zeros_hbm, acc.at[pl.ds(sid * TPR, TPR)])
    plsc.subcore_barrier()
    wid = cid * 16 + sid
    base = wid * (EROWS // 32)

    def w(i, _):
        pltpu.sync_copy(dst2d.at[pl.ds(base + i * 16, 16)], dstv)
        for j in range(16):
            pltpu.sync_copy(onesv, acc.at[dstv.at[j]], add=True)
        return 0

    lax.fori_loop(0, 25, w, 0)
    plsc.subcore_barrier()
    pltpu.sync_copy(acc.at[pl.ds(sid * TPR, TPR)],
                    out.at[cid].at[pl.ds(sid * TPR, TPR)])


def _make_agg(C):
    @functools.partial(
        pl.kernel, mesh=_mesh, compiler_params=_sc_params,
        out_type=jax.ShapeDtypeStruct((NPAD, C * 16), jnp.float32),
        scratch_types=[
            pltpu.VMEM_SHARED((NP16, 16), jnp.float32),
            pltpu.VMEM((3, 4, 128), jnp.int32),
            pltpu.VMEM((3, 4, 128), jnp.int32),
            pltpu.VMEM((3, 512, 16), jnp.float32),
            pltpu.SemaphoreType.DMA,
            pltpu.SemaphoreType.DMA,
            pltpu.SemaphoreType.DMA,
            pltpu.SemaphoreType.DMA,
            pltpu.SemaphoreType.DMA,
            pltpu.SemaphoreType.DMA,
        ],
    )
    def _agg(ycm, src2d, dst2d, zeros_hbm, out,
             acc, svb, dvb, rb, sg0, sg1, sg2, ss0, ss1, ss2):
        cid = lax.axis_index("c")
        sid = lax.axis_index("s")
        nsl = pl.ds(sid * TPR, TPR)
        base = sid * (EROWS // 16)
        NW = (EROWS // 16) // 4
        sg = [sg0, sg1, sg2]
        ss = [ss0, ss1, ss2]

        def chunk(k, _):
            @pl.when((k % 2) == cid)
            def _():
                pltpu.sync_copy(zeros_hbm, acc.at[nsl])
                plsc.subcore_barrier()

                def lidx(w, b):
                    off = base + w * 4
                    pltpu.sync_copy(src2d.at[pl.ds(off, 4)], svb.at[b])
                    pltpu.sync_copy(dst2d.at[pl.ds(off, 4)], dvb.at[b])

                def fire_g(b):
                    for j in range(4):
                        pltpu.async_copy(ycm.at[k].at[svb.at[b].at[j]],
                                         rb.at[b].at[pl.ds(j * 128, 128)],
                                         sg[b])

                def drain_g(b):
                    for j in range(4):
                        pltpu.make_async_copy(
                            ycm.at[k].at[svb.at[b].at[j]],
                            rb.at[b].at[pl.ds(j * 128, 128)], sg[b]).wait()

                def fire_s(b):
                    for j in range(4):
                        pltpu.async_copy(rb.at[b].at[pl.ds(j * 128, 128)],
                                         acc.at[dvb.at[b].at[j]], ss[b],
                                         add=True)

                def drain_s(b):
                    for j in range(4):
                        pltpu.make_async_copy(
                            rb.at[b].at[pl.ds(j * 128, 128)],
                            acc.at[dvb.at[b].at[j]], ss[b]).wait()

                def step(w, u, guard_first):
                    a, n = u % 3, (u + 1) % 3
                    if guard_first:
                        @pl.when(w >= 2)
                        def _():
                            drain_s(n)
                    else:
                        drain_s(n)
                    lidx(w + 1, n)
                    fire_g(n)
                    drain_g(a)
                    fire_s(a)

                lidx(0, 0)
                fire_g(0)

                def body(i, _):
                    w = 3 * i
                    step(w, 0, True)
                    step(w + 1, 1, True)
                    step(w + 2, 2, False)
                    return 0

                lax.fori_loop(0, (NW - 2) // 3, body, 0)
                drain_s(1)
                lidx(NW - 1, 1)
                fire_g(1)
                drain_g(0)
                fire_s(0)
                drain_s(2)
                drain_g(1)
                fire_s(1)
                drain_s(0)
                drain_s(1)
                plsc.subcore_barrier()
                pltpu.sync_copy(acc.at[nsl],
                                out.at[nsl, pl.ds(k * 16, 16)])

                @pl.when(sid == 0)
                def _():
                    pltpu.sync_copy(
                        zeros_hbm.at[pl.ds(0, NPAD - NP16)],
                        out.at[pl.ds(NP16, NPAD - NP16), pl.ds(k * 16, 16)])
            return 0

        lax.fori_loop(0, C, chunk, 0)

    return _agg


def _prep_body(d0, d1, xp, dinv_ref, xs_ref):
    deg = d0[...] + d1[...] + 1.0
    dv = lax.rsqrt(deg)
    dinv_ref[...] = dv
    xs_ref[...] = xp[...] * dv


def _l1_body(a1, xs, dinv, w1, b1, w2, y2_ref):
    pre = dinv[...] * (a1[...] + xs[...])
    h1 = jnp.maximum(jnp.dot(pre, w1[...],
                             preferred_element_type=jnp.float32) + b1[...], 0.0)
    y2_ref[...] = jnp.dot(h1, w2[...],
                          preferred_element_type=jnp.float32) * dinv[...]


def _l2_body(a2, y2, dinv, b2, w3, y3_ref):
    h2 = jnp.maximum(dinv[...] * (a2[...] + y2[...]) + b2[...], 0.0)
    y3_ref[...] = jnp.dot(h2, w3[...],
                          preferred_element_type=jnp.float32) * dinv[...]


def _l3_body(a3, y3, dinv, b3, bat, out_ref):
    @pl.when(pl.program_id(0) == 0)
    def _():
        out_ref[...] = jnp.full((GG, 304), -jnp.inf, jnp.float32)

    h3 = jnp.maximum(dinv[...] * (a3[...] + y3[...]) + b3[...], 0.0)
    b = bat[...]
    g0 = jnp.min(b)
    g1 = jnp.minimum(jnp.max(b), 63)

    def body(g, _):
        m = (b == g)
        contrib = jnp.max(jnp.where(m, h3, -jnp.inf), axis=0, keepdims=True)
        cur = out_ref[pl.ds(g, 1), :]
        out_ref[pl.ds(g, 1), :] = jnp.maximum(cur, contrib)
        return 0

    lax.fori_loop(g0, g1 + 1, body, 0)


def _head_body(gm, w1, b1, w2, b2, w3, b3, out_ref):
    g = jnp.maximum(gm[...], 0.0)
    h = jnp.maximum(jnp.dot(g, w1[...],
                            preferred_element_type=jnp.float32) + b1[...], 0.0)
    h = jnp.maximum(jnp.dot(h, w2[...],
                            preferred_element_type=jnp.float32) + b2[...], 0.0)
    lg = jnp.dot(h, w3[...], preferred_element_type=jnp.float32) + b3[...]
    m = jnp.max(lg, axis=0, keepdims=True)
    e = jnp.exp(lg - m)
    out_ref[...] = e / jnp.sum(e, axis=0, keepdims=True)


def _row_spec(w):
    return pl.BlockSpec((RB, w), lambda i: (i, 0))


def _full_spec(shape):
    return pl.BlockSpec(shape, lambda i: tuple(0 for _ in shape))


def _tc_call(body, ins, in_specs, out_shape, out_spec):
    return pl.pallas_call(
        body,
        grid=(98,),
        in_specs=in_specs,
        out_specs=out_spec,
        out_shape=out_shape,
        compiler_params=pltpu.CompilerParams(
            dimension_semantics=("arbitrary",)),
    )(*ins)


def _to_cm(y, C):
    t = y[:NN, :C * 16].reshape(NN, C, 16).transpose(1, 0, 2)
    return jnp.pad(t, ((0, 0), (0, NP16 - NN), (0, 0)))


def kernel(x, edge_index, batch, W1, b1, W2, b2, W3, b3,
           L1w, L1b, L2w, L2b, L3w, L3b):
    f32 = jnp.float32
    src, dst = edge_index[0], edge_index[1]
    npad_e = EP - EE
    padi = NN + (jnp.arange(npad_e, dtype=jnp.int32) % 16)
    src2d = jnp.concatenate([src, padi]).reshape(EROWS, 128)
    dst2d = jnp.concatenate([dst, padi]).reshape(EROWS, 128)

    ones_h = jnp.ones((128, 16), f32)
    zeros_h = jnp.zeros((TPR, 16), f32)

    degp = _deg_call(dst2d, ones_h, zeros_h)
    d0 = jnp.pad(degp[0, :NN, 0:1], ((0, NPAD - NN), (0, 0)))
    d1 = jnp.pad(degp[1, :NN, 0:1], ((0, NPAD - NN), (0, 0)))

    xp = jnp.pad(x, ((0, NPAD - NN), (0, 32 - x.shape[1])))
    dinv, xs = _tc_call(
        _prep_body,
        (d0, d1, xp),
        [_row_spec(1), _row_spec(1), _row_spec(32)],
        (jax.ShapeDtypeStruct((NPAD, 1), f32),
         jax.ShapeDtypeStruct((NPAD, 32), f32)),
        (_row_spec(1), _row_spec(32)),
    )

    a1 = _make_agg(2)(_to_cm(xs, 2), src2d, dst2d, zeros_h)

    W1p = jnp.zeros((32, 512), f32).at[:19, :500].set(W1)
    b1p = jnp.zeros((1, 512), f32).at[0, :500].set(b1)
    W2p = jnp.zeros((512, 400), f32).at[:500, :].set(W2)
    y2 = _tc_call(
        _l1_body,
        (a1, xs, dinv, W1p, b1p, W2p),
        [_row_spec(32), _row_spec(32), _row_spec(1),
         _full_spec((32, 512)), _full_spec((1, 512)), _full_spec((512, 400))],
        jax.ShapeDtypeStruct((NPAD, 400), f32),
        _row_spec(400),
    )

    a2 = _make_agg(25)(_to_cm(y2, 25), src2d, dst2d, zeros_h)

    b2p = b2[None, :]
    W3p = jnp.zeros((400, 304), f32).at[:, :300].set(W3)
    y3 = _tc_call(
        _l2_body,
        (a2, y2, dinv, b2p, W3p),
        [_row_spec(400), _row_spec(400), _row_spec(1),
         _full_spec((1, 400)), _full_spec((400, 304))],
        jax.ShapeDtypeStruct((NPAD, 304), f32),
        _row_spec(304),
    )

    a3 = _make_agg(19)(_to_cm(y3, 19), src2d, dst2d, zeros_h)

    b3p = jnp.zeros((1, 304), f32).at[0, :300].set(b3)
    batp = jnp.pad(batch, (0, NPAD - NN), constant_values=GG)[:, None]
    gmax = _tc_call(
        _l3_body,
        (a3, y3, dinv, b3p, batp),
        [_row_spec(304), _row_spec(304), _row_spec(1),
         _full_spec((1, 304)), _row_spec(1)],
        jax.ShapeDtypeStruct((GG, 304), f32),
        _full_spec((GG, 304)),
    )

    L1wp = jnp.zeros((304, 256), f32).at[:300, :200].set(L1w)
    L1bp = jnp.zeros((1, 256), f32).at[0, :200].set(L1b)
    L2wp = jnp.zeros((256, 128), f32).at[:200, :100].set(L2w)
    L2bp = jnp.zeros((1, 128), f32).at[0, :100].set(L2b)
    L3wp = jnp.zeros((128, 128), f32).at[:100, :6].set(L3w)
    L3bp = jnp.zeros((1, 128), f32).at[0, :6].set(L3b)

    out = pl.pallas_call(
        _head_body,
        out_shape=jax.ShapeDtypeStruct((GG, 128), f32),
    )(gmax, L1wp, L1bp, L2wp, L2bp, L3wp, L3bp)

    return out[:, :6]


# speedup vs baseline: 4.8542x; 1.0455x over previous
import functools
import jax
import jax.numpy as jnp
from jax import lax
from jax.experimental import pallas as pl
from jax.experimental.pallas import tpu as pltpu
from jax.experimental.pallas import tpu_sc as plsc

NN = 100000
EE = 1600000
GG = 64
RB = 1024
NPAD = 98 * RB
NP16 = NN + 96
EP = 1638400
EROWS = EP // 128
TPR = NP16 // 16

_mesh = plsc.VectorSubcoreMesh(core_axis_name="c", subcore_axis_name="s")


_sc_params = pltpu.CompilerParams(use_tc_tiling_on_sc=False)


@functools.partial(
    pl.kernel, mesh=_mesh, compiler_params=_sc_params,
    out_type=jax.ShapeDtypeStruct((2, NP16, 16), jnp.float32),
    scratch_types=[
        pltpu.VMEM_SHARED((NP16, 16), jnp.float32),
        pltpu.VMEM((16, 128), jnp.int32),
        pltpu.VMEM((128, 16), jnp.float32),
    ],
)
def _deg_call(dst2d, ones_hbm, zeros_hbm, out, acc, dstv, onesv):
    cid = lax.axis_index("c")
    sid = lax.axis_index("s")
    pltpu.sync_copy(ones_hbm, onesv)
    pltpu.sync_copy(zeros_hbm, acc.at[pl.ds(sid * TPR, TPR)])
    plsc.subcore_barrier()
    wid = cid * 16 + sid
    base = wid * (EROWS // 32)

    def w(i, _):
        pltpu.sync_copy(dst2d.at[pl.ds(base + i * 16, 16)], dstv)
        for j in range(16):
            pltpu.sync_copy(onesv, acc.at[dstv.at[j]], add=True)
        return 0

    lax.fori_loop(0, 25, w, 0)
    plsc.subcore_barrier()
    pltpu.sync_copy(acc.at[pl.ds(sid * TPR, TPR)],
                    out.at[cid].at[pl.ds(sid * TPR, TPR)])


def _make_agg(C):
    @functools.partial(
        pl.kernel, mesh=_mesh, compiler_params=_sc_params,
        out_type=jax.ShapeDtypeStruct((NPAD, C * 16), jnp.float32),
        scratch_types=[
            pltpu.VMEM_SHARED((NP16, 16), jnp.float32),
            pltpu.VMEM((3, 4, 256), jnp.int32),
            pltpu.VMEM((3, 512, 16), jnp.float32),
            pltpu.SemaphoreType.DMA,
            pltpu.SemaphoreType.DMA,
            pltpu.SemaphoreType.DMA,
            pltpu.SemaphoreType.DMA,
            pltpu.SemaphoreType.DMA,
            pltpu.SemaphoreType.DMA,
        ],
    )
    def _agg(ycm, e2d, zeros_hbm, out,
             acc, evb, rb, sg0, sg1, sg2, ss0, ss1, ss2):
        cid = lax.axis_index("c")
        sid = lax.axis_index("s")
        nsl = pl.ds(sid * TPR, TPR)
        base = sid * (EROWS // 16)
        NW = (EROWS // 16) // 4
        sg = [sg0, sg1, sg2]
        ss = [ss0, ss1, ss2]

        def chunk(k, _):
            @pl.when((k % 2) == cid)
            def _():
                pltpu.sync_copy(zeros_hbm, acc.at[nsl])
                plsc.subcore_barrier()

                def lidx(w, b):
                    off = base + w * 4
                    pltpu.sync_copy(e2d.at[pl.ds(off, 4)], evb.at[b])

                def fire_g(b):
                    for j in range(4):
                        pltpu.async_copy(
                            ycm.at[k].at[evb.at[b].at[j, pl.ds(0, 128)]],
                            rb.at[b].at[pl.ds(j * 128, 128)], sg[b])

                def drain_g(b):
                    for j in range(4):
                        pltpu.make_async_copy(
                            ycm.at[k].at[evb.at[b].at[j, pl.ds(0, 128)]],
                            rb.at[b].at[pl.ds(j * 128, 128)], sg[b]).wait()

                def fire_s(b):
                    for j in range(4):
                        pltpu.async_copy(
                            rb.at[b].at[pl.ds(j * 128, 128)],
                            acc.at[evb.at[b].at[j, pl.ds(128, 128)]], ss[b],
                            add=True)

                def drain_s(b):
                    for j in range(4):
                        pltpu.make_async_copy(
                            rb.at[b].at[pl.ds(j * 128, 128)],
                            acc.at[evb.at[b].at[j, pl.ds(128, 128)]],
                            ss[b]).wait()

                def step(w, u, guard_first):
                    a, n = u % 3, (u + 1) % 3
                    if guard_first:
                        @pl.when(w >= 2)
                        def _():
                            drain_s(n)
                    else:
                        drain_s(n)
                    lidx(w + 1, n)
                    fire_g(n)
                    drain_g(a)
                    fire_s(a)

                lidx(0, 0)
                fire_g(0)

                def body(i, _):
                    w = 3 * i
                    step(w, 0, True)
                    step(w + 1, 1, True)
                    step(w + 2, 2, False)
                    return 0

                lax.fori_loop(0, (NW - 2) // 3, body, 0)
                drain_s(1)
                lidx(NW - 1, 1)
                fire_g(1)
                drain_g(0)
                fire_s(0)
                drain_s(2)
                drain_g(1)
                fire_s(1)
                drain_s(0)
                drain_s(1)
                plsc.subcore_barrier()
                pltpu.sync_copy(acc.at[nsl],
                                out.at[nsl, pl.ds(k * 16, 16)])

                @pl.when(sid == 0)
                def _():
                    pltpu.sync_copy(
                        zeros_hbm.at[pl.ds(0, NPAD - NP16)],
                        out.at[pl.ds(NP16, NPAD - NP16), pl.ds(k * 16, 16)])
            return 0

        lax.fori_loop(0, C, chunk, 0)

    return _agg


def _prep_body(d0, d1, xp, dinv_ref, xs_ref):
    deg = d0[...] + d1[...] + 1.0
    dv = lax.rsqrt(deg)
    dinv_ref[...] = dv
    xs_ref[...] = xp[...] * dv


def _l1_body(a1, xs, dinv, w1, b1, w2, y2_ref):
    pre = dinv[...] * (a1[...] + xs[...])
    h1 = jnp.maximum(jnp.dot(pre, w1[...],
                             preferred_element_type=jnp.float32) + b1[...], 0.0)
    y2_ref[...] = jnp.dot(h1, w2[...],
                          preferred_element_type=jnp.float32) * dinv[...]


def _l2_body(a2, y2, dinv, b2, w3, y3_ref):
    h2 = jnp.maximum(dinv[...] * (a2[...] + y2[...]) + b2[...], 0.0)
    y3_ref[...] = jnp.dot(h2, w3[...],
                          preferred_element_type=jnp.float32) * dinv[...]


def _l3_body(a3, y3, dinv, b3, bat, out_ref):
    @pl.when(pl.program_id(0) == 0)
    def _():
        out_ref[...] = jnp.full((GG, 304), -jnp.inf, jnp.float32)

    h3 = jnp.maximum(dinv[...] * (a3[...] + y3[...]) + b3[...], 0.0)
    b = bat[...]
    g0 = jnp.min(b)
    g1 = jnp.minimum(jnp.max(b), 63)

    def body(g, _):
        m = (b == g)
        contrib = jnp.max(jnp.where(m, h3, -jnp.inf), axis=0, keepdims=True)
        cur = out_ref[pl.ds(g, 1), :]
        out_ref[pl.ds(g, 1), :] = jnp.maximum(cur, contrib)
        return 0

    lax.fori_loop(g0, g1 + 1, body, 0)


def _head_body(gm, w1, b1, w2, b2, w3, b3, out_ref):
    g = jnp.maximum(gm[...], 0.0)
    h = jnp.maximum(jnp.dot(g, w1[...],
                            preferred_element_type=jnp.float32) + b1[...], 0.0)
    h = jnp.maximum(jnp.dot(h, w2[...],
                            preferred_element_type=jnp.float32) + b2[...], 0.0)
    lg = jnp.dot(h, w3[...], preferred_element_type=jnp.float32) + b3[...]
    m = jnp.max(lg, axis=0, keepdims=True)
    e = jnp.exp(lg - m)
    out_ref[...] = e / jnp.sum(e, axis=0, keepdims=True)


def _row_spec(w):
    return pl.BlockSpec((RB, w), lambda i: (i, 0))


def _full_spec(shape):
    return pl.BlockSpec(shape, lambda i: tuple(0 for _ in shape))


def _tc_call(body, ins, in_specs, out_shape, out_spec):
    return pl.pallas_call(
        body,
        grid=(98,),
        in_specs=in_specs,
        out_specs=out_spec,
        out_shape=out_shape,
        compiler_params=pltpu.CompilerParams(
            dimension_semantics=("arbitrary",)),
    )(*ins)


def _to_cm(y, C):
    t = y[:NN, :C * 16].reshape(NN, C, 16).transpose(1, 0, 2)
    return jnp.pad(t, ((0, 0), (0, NP16 - NN), (0, 0)))


def kernel(x, edge_index, batch, W1, b1, W2, b2, W3, b3,
           L1w, L1b, L2w, L2b, L3w, L3b):
    f32 = jnp.float32
    src, dst = edge_index[0], edge_index[1]
    npad_e = EP - EE
    padi = NN + (jnp.arange(npad_e, dtype=jnp.int32) % 16)
    src2d = jnp.concatenate([src, padi]).reshape(EROWS, 128)
    dst2d = jnp.concatenate([dst, padi]).reshape(EROWS, 128)
    e2d = jnp.concatenate([src2d, dst2d], axis=1)

    ones_h = jnp.ones((128, 16), f32)
    zeros_h = jnp.zeros((TPR, 16), f32)

    degp = _deg_call(dst2d, ones_h, zeros_h)
    d0 = jnp.pad(degp[0, :NN, 0:1], ((0, NPAD - NN), (0, 0)))
    d1 = jnp.pad(degp[1, :NN, 0:1], ((0, NPAD - NN), (0, 0)))

    xp = jnp.pad(x, ((0, NPAD - NN), (0, 32 - x.shape[1])))
    dinv, xs = _tc_call(
        _prep_body,
        (d0, d1, xp),
        [_row_spec(1), _row_spec(1), _row_spec(32)],
        (jax.ShapeDtypeStruct((NPAD, 1), f32),
         jax.ShapeDtypeStruct((NPAD, 32), f32)),
        (_row_spec(1), _row_spec(32)),
    )

    a1 = _make_agg(2)(_to_cm(xs, 2), e2d, zeros_h)

    W1p = jnp.zeros((32, 512), f32).at[:19, :500].set(W1)
    b1p = jnp.zeros((1, 512), f32).at[0, :500].set(b1)
    W2p = jnp.zeros((512, 400), f32).at[:500, :].set(W2)
    y2 = _tc_call(
        _l1_body,
        (a1, xs, dinv, W1p, b1p, W2p),
        [_row_spec(32), _row_spec(32), _row_spec(1),
         _full_spec((32, 512)), _full_spec((1, 512)), _full_spec((512, 400))],
        jax.ShapeDtypeStruct((NPAD, 400), f32),
        _row_spec(400),
    )

    a2 = _make_agg(25)(_to_cm(y2, 25), e2d, zeros_h)

    b2p = b2[None, :]
    W3p = jnp.zeros((400, 304), f32).at[:, :300].set(W3)
    y3 = _tc_call(
        _l2_body,
        (a2, y2, dinv, b2p, W3p),
        [_row_spec(400), _row_spec(400), _row_spec(1),
         _full_spec((1, 400)), _full_spec((400, 304))],
        jax.ShapeDtypeStruct((NPAD, 304), f32),
        _row_spec(304),
    )

    a3 = _make_agg(19)(_to_cm(y3, 19), e2d, zeros_h)

    b3p = jnp.zeros((1, 304), f32).at[0, :300].set(b3)
    batp = jnp.pad(batch, (0, NPAD - NN), constant_values=GG)[:, None]
    gmax = _tc_call(
        _l3_body,
        (a3, y3, dinv, b3p, batp),
        [_row_spec(304), _row_spec(304), _row_spec(1),
         _full_spec((1, 304)), _row_spec(1)],
        jax.ShapeDtypeStruct((GG, 304), f32),
        _full_spec((GG, 304)),
    )

    L1wp = jnp.zeros((304, 256), f32).at[:300, :200].set(L1w)
    L1bp = jnp.zeros((1, 256), f32).at[0, :200].set(L1b)
    L2wp = jnp.zeros((256, 128), f32).at[:200, :100].set(L2w)
    L2bp = jnp.zeros((1, 128), f32).at[0, :100].set(L2b)
    L3wp = jnp.zeros((128, 128), f32).at[:100, :6].set(L3w)
    L3bp = jnp.zeros((1, 128), f32).at[0, :6].set(L3b)

    out = pl.pallas_call(
        _head_body,
        out_shape=jax.ShapeDtypeStruct((GG, 128), f32),
    )(gmax, L1wp, L1bp, L2wp, L2bp, L3wp, L3bp)

    return out[:, :6]
